# baseline (speedup 1.0000x reference)
import functools

import jax
import jax.numpy as jnp
from jax import lax
from jax.experimental import pallas as pl
from jax.experimental.pallas import tpu as pltpu
from jax.experimental.pallas import tpu_sc as plsc

N_NODES = 10000
E = 160000
D_IN = 256
D_OUT = 256

NC = 2
NS = 16

N_PAD = 10112
TRASH = N_NODES
CHUNK = 96
NBUF = 4
EDGES_PER_TILE = E // NS
FULL = EDGES_PER_TILE // CHUNK
NCH = FULL + 1
REM_OFF = EDGES_PER_TILE - CHUNK
ROWS_PER_TILE = N_PAD // NS


def _seg_sum_sc(feat, dst4):
    mesh = plsc.VectorSubcoreMesh(core_axis_name="c", subcore_axis_name="s")

    @functools.partial(
        pl.kernel,
        mesh=mesh,
        out_type=jax.ShapeDtypeStruct((NC, N_PAD, 128), jnp.float32),
        scratch_types=[
            pltpu.VMEM((NBUF, CHUNK), jnp.int32),
            pltpu.VMEM((NBUF, CHUNK, 128), jnp.float32),
            pltpu.VMEM_SHARED((N_PAD, 128), jnp.float32),
            pltpu.SemaphoreType.DMA,
            pltpu.SemaphoreType.DMA,
            pltpu.SemaphoreType.DMA,
            pltpu.SemaphoreType.DMA,
            pltpu.SemaphoreType.DMA,
            pltpu.SemaphoreType.DMA,
            pltpu.SemaphoreType.DMA,
            pltpu.SemaphoreType.DMA,
            pltpu.SemaphoreType.DMA,
            pltpu.SemaphoreType.DMA,
            pltpu.SemaphoreType.DMA,
            pltpu.SemaphoreType.DMA,
        ],
    )
    def seg_sum(
        feat_hbm, dst_hbm, out_hbm, idx_v, bufs, shared,
        gsem0, gsem1, gsem2, gsem3,
        ssem0, ssem1, ssem2, ssem3,
        isem0, isem1, isem2, isem3,
    ):
        c = lax.axis_index("c")
        s = lax.axis_index("s")
        gsems = (gsem0, gsem1, gsem2, gsem3)
        ssems = (ssem0, ssem1, ssem2, ssem3)
        isems = (isem0, isem1, isem2, isem3)

        def start_gather(j, b):
            e0 = s * EDGES_PER_TILE + jnp.where(j < FULL, j * CHUNK, REM_OFF)
            pltpu.async_copy(
                feat_hbm.at[pl.ds(e0, CHUNK), pl.ds(c * 128, 128)],
                bufs.at[b],
                gsems[b],
            )

        def wait_gather(b):
            pltpu.make_async_copy(
                feat_hbm.at[pl.ds(0, CHUNK), pl.ds(0, 128)], bufs.at[b], gsems[b]
            ).wait()

        def start_idx(j, b):
            pltpu.async_copy(dst_hbm.at[s, j, 0], idx_v.at[b], isems[b])

        def wait_idx(b):
            pltpu.make_async_copy(
                dst_hbm.at[0, 0, 0], idx_v.at[b], isems[b]
            ).wait()

        def start_scatter(b):
            pltpu.async_copy(
                bufs.at[b], shared.at[idx_v.at[b]], ssems[b], add=True
            )

        def wait_scatter(b):
            pltpu.make_async_copy(
                bufs.at[b], shared.at[idx_v.at[b]], ssems[b]
            ).wait()

        for p in range(NBUF - 1):
            start_gather(p, p)
            start_idx(p, p)

        def zrow(i, _):
            for k in range(8):
                bufs[NBUF - 1, i, pl.ds(k * 16, 16)] = jnp.zeros(
                    (16,), jnp.float32
                )
            return 0

        lax.fori_loop(0, CHUNK, zrow, 0)

        for t in range(ROWS_PER_TILE // CHUNK):
            pltpu.sync_copy(
                bufs.at[NBUF - 1, pl.ds(0, CHUNK)],
                shared.at[pl.ds(s * ROWS_PER_TILE + t * CHUNK, CHUNK)],
            )
        _TAIL = ROWS_PER_TILE % CHUNK
        if _TAIL:
            pltpu.sync_copy(
                bufs.at[NBUF - 1, pl.ds(0, _TAIL)],
                shared.at[
                    pl.ds(s * ROWS_PER_TILE + ROWS_PER_TILE - _TAIL, _TAIL)
                ],
            )

        plsc.subcore_barrier()


        def chunk_step(j, b):
            bp = (b + NBUF - 1) % NBUF

            @pl.when(j < NCH)
            def _():
                wait_gather(b)
                wait_idx(b)

                @pl.when(j >= 1)
                def _():
                    wait_scatter(bp)

                @pl.when(j + NBUF - 1 < NCH)
                def _():
                    start_gather(j + NBUF - 1, bp)
                    start_idx(j + NBUF - 1, bp)

                start_scatter(b)

        def quad(i, _):
            j0 = NBUF * i
            for b in range(NBUF):
                chunk_step(j0 + b, b)
            return 0

        lax.fori_loop(0, (NCH + NBUF - 1) // NBUF, quad, 0)
        wait_scatter((NCH - 1) % NBUF)

        plsc.subcore_barrier()

        pltpu.sync_copy(
            shared.at[pl.ds(s * ROWS_PER_TILE, ROWS_PER_TILE)],
            out_hbm.at[c, pl.ds(s * ROWS_PER_TILE, ROWS_PER_TILE)],
        )

    return seg_sum(feat, dst4)


BN = 2000


def _mm_body(h_ref, w_ref, b_ref, o_ref):
    h0 = h_ref[0]
    h1 = h_ref[1]
    w = w_ref[...]
    dn = (((1,), (1,)), ((), ()))
    acc = lax.dot_general(h0, w[:, :128], dn, preferred_element_type=jnp.float32)
    acc = acc + lax.dot_general(
        h1, w[:, 128:], dn, preferred_element_type=jnp.float32
    )
    o_ref[...] = acc + b_ref[...]


def _linear_tc(h2, W, b2):
    return pl.pallas_call(
        _mm_body,
        grid=(N_NODES // BN,),
        in_specs=[
            pl.BlockSpec((NC, BN, 128), lambda i: (0, i, 0)),
            pl.BlockSpec((D_OUT, D_IN), lambda i: (0, 0)),
            pl.BlockSpec((1, D_OUT), lambda i: (0, 0)),
        ],
        out_specs=pl.BlockSpec((BN, D_OUT), lambda i: (i, 0)),
        out_shape=jax.ShapeDtypeStruct((N_NODES, D_OUT), jnp.float32),
    )(h2, W, b2)


def kernel(features, edge_index, W, b):
    dst = edge_index[1].astype(jnp.int32).reshape(NS, EDGES_PER_TILE)
    full = dst[:, : FULL * CHUNK].reshape(NS, FULL, CHUNK)
    rem = dst[:, REM_OFF:]
    lane = jnp.arange(CHUNK, dtype=jnp.int32)
    rem = jnp.where(lane >= CHUNK - (EDGES_PER_TILE - FULL * CHUNK), rem, TRASH)
    dst4 = jnp.concatenate([full, rem[:, None, :]], axis=1).reshape(
        NS, NCH, 1, CHUNK
    )
    h2 = _seg_sum_sc(features, dst4)
    out = _linear_tc(h2, W, b.reshape(1, D_OUT))
    return out


# speedup vs baseline: 1.0122x; 1.0122x over previous
import functools

import jax
import jax.numpy as jnp
from jax import lax
from jax.experimental import pallas as pl
from jax.experimental.pallas import tpu as pltpu
from jax.experimental.pallas import tpu_sc as plsc

N_NODES = 10000
E = 160000
D_IN = 256
D_OUT = 256

NC = 2
NS = 16

N_PAD = 10240
TRASH = N_NODES
CHUNK = 88
NBUF = 3
EDGES_PER_TILE = E // NS
FULL = EDGES_PER_TILE // CHUNK
NCH = FULL + 1
REM_OFF = EDGES_PER_TILE - CHUNK
ROWS_PER_TILE = N_PAD // NS
ZR = 80


def _seg_sum_sc(feat, dst3):
    mesh = plsc.VectorSubcoreMesh(core_axis_name="c", subcore_axis_name="s")

    @functools.partial(
        pl.kernel,
        mesh=mesh,
        out_type=jax.ShapeDtypeStruct((NC, N_PAD, 128), jnp.float32),
        scratch_types=[
            pltpu.VMEM((NCH, CHUNK), jnp.int32),
            pltpu.VMEM((NBUF, CHUNK, 128), jnp.float32),
            pltpu.VMEM_SHARED((N_PAD, 128), jnp.float32),
            pltpu.SemaphoreType.DMA,
            pltpu.SemaphoreType.DMA,
            pltpu.SemaphoreType.DMA,
            pltpu.SemaphoreType.DMA,
            pltpu.SemaphoreType.DMA,
            pltpu.SemaphoreType.DMA,
            pltpu.SemaphoreType.DMA,
        ],
    )
    def seg_sum(
        feat_hbm, dst_hbm, out_hbm, idx_v, bufs, shared,
        gsem0, gsem1, gsem2, ssem0, ssem1, ssem2, isem
    ):
        c = lax.axis_index("c")
        s = lax.axis_index("s")
        gsems = (gsem0, gsem1, gsem2)
        ssems = (ssem0, ssem1, ssem2)

        def start_gather(j, b):
            e0 = s * EDGES_PER_TILE + jnp.where(j < FULL, j * CHUNK, REM_OFF)
            pltpu.async_copy(
                feat_hbm.at[pl.ds(e0, CHUNK), pl.ds(c * 128, 128)],
                bufs.at[b],
                gsems[b],
            )

        def wait_gather(b):
            pltpu.make_async_copy(
                feat_hbm.at[pl.ds(0, CHUNK), pl.ds(0, 128)], bufs.at[b], gsems[b]
            ).wait()

        def start_scatter(j, b):
            pltpu.async_copy(
                bufs.at[b], shared.at[idx_v.at[j]], ssems[b], add=True
            )

        def wait_scatter(b):
            pltpu.make_async_copy(
                bufs.at[b], shared.at[idx_v.at[0]], ssems[b]
            ).wait()

        start_gather(0, 0)
        start_gather(1, 1)
        pltpu.async_copy(dst_hbm.at[s], idx_v, isem)

        def zrow(i, _):
            for k in range(8):
                bufs[2, i, pl.ds(k * 16, 16)] = jnp.zeros((16,), jnp.float32)
            return 0

        lax.fori_loop(0, ZR, zrow, 0)

        def zcp(t, _):
            pltpu.sync_copy(
                bufs.at[2, pl.ds(0, ZR)],
                shared.at[pl.ds(s * ROWS_PER_TILE + t * ZR, ZR)],
            )
            return 0

        lax.fori_loop(0, ROWS_PER_TILE // ZR, zcp, 0)

        pltpu.make_async_copy(dst_hbm.at[s], idx_v, isem).wait()

        plsc.subcore_barrier()


        def chunk_step(j, b):
            bp = (b + 2) % NBUF
            wait_gather(b)

            @pl.when(j >= 1)
            def _():
                wait_scatter(bp)

            @pl.when(j + 2 < NCH)
            def _():
                start_gather(j + 2, bp)

            start_scatter(j, b)

        def triple(i, _):
            j0 = NBUF * i
            for b in range(NBUF):
                chunk_step(j0 + b, b)
            return 0

        lax.fori_loop(0, NCH // NBUF, triple, 0)
        wait_scatter((NCH - 1) % NBUF)

        plsc.subcore_barrier()

        pltpu.sync_copy(
            shared.at[pl.ds(s * ROWS_PER_TILE, ROWS_PER_TILE)],
            out_hbm.at[c, pl.ds(s * ROWS_PER_TILE, ROWS_PER_TILE)],
        )

    return seg_sum(feat, dst3)


BN = 2000


def _mm_body(h_ref, w_ref, b_ref, o_ref):
    h0 = h_ref[0]
    h1 = h_ref[1]
    w = w_ref[...]
    dn = (((1,), (1,)), ((), ()))
    acc = lax.dot_general(h0, w[:, :128], dn, preferred_element_type=jnp.float32)
    acc = acc + lax.dot_general(
        h1, w[:, 128:], dn, preferred_element_type=jnp.float32
    )
    o_ref[...] = acc + b_ref[...]


def _linear_tc(h2, W, b2):
    return pl.pallas_call(
        _mm_body,
        grid=(N_NODES // BN,),
        in_specs=[
            pl.BlockSpec((NC, BN, 128), lambda i: (0, i, 0)),
            pl.BlockSpec((D_OUT, D_IN), lambda i: (0, 0)),
            pl.BlockSpec((1, D_OUT), lambda i: (0, 0)),
        ],
        out_specs=pl.BlockSpec((BN, D_OUT), lambda i: (i, 0)),
        out_shape=jax.ShapeDtypeStruct((N_NODES, D_OUT), jnp.float32),
    )(h2, W, b2)


def kernel(features, edge_index, W, b):
    dst = edge_index[1].astype(jnp.int32).reshape(NS, EDGES_PER_TILE)
    full = dst[:, : FULL * CHUNK].reshape(NS, FULL, CHUNK)
    rem = dst[:, REM_OFF:]
    lane = jnp.arange(CHUNK, dtype=jnp.int32)
    rem = jnp.where(lane >= CHUNK - (EDGES_PER_TILE - FULL * CHUNK), rem, TRASH)
    dst3 = jnp.concatenate([full, rem[:, None, :]], axis=1)
    h2 = _seg_sum_sc(features, dst3)
    out = _linear_tc(h2, W, b.reshape(1, D_OUT))
    return out
